# baseline (speedup 1.0000x reference)
import functools

import jax
import jax.numpy as jnp
from jax import lax
from jax.experimental import pallas as pl
from jax.experimental.pallas import tpu as pltpu
from jax.experimental.pallas import tpu_sc as plsc

EMBED_DIM = 32
BT = 128
ET = EMBED_DIM // 8


def _sc_gather(table, idx3, H, n_bt, NC, NS):
    NW = NC * NS
    B = n_bt * BT
    bt_per_w = n_bt // NW
    n_units = H * bt_per_w
    mesh = plsc.VectorSubcoreMesh(core_axis_name="c", subcore_axis_name="s")

    @functools.partial(
        pl.kernel,
        mesh=mesh,
        out_type=jax.ShapeDtypeStruct((H, ET, n_bt, 8, BT), jnp.float32),
        scratch_types=[
            pltpu.VMEM((H, bt_per_w, BT), jnp.int32),
            pltpu.VMEM((BT, EMBED_DIM), jnp.float32),
            pltpu.VMEM((BT, EMBED_DIM), jnp.float32),
            pltpu.VMEM((ET, 8, BT), jnp.float32),
            pltpu.VMEM((ET, 8, BT), jnp.float32),
            pltpu.SemaphoreType.DMA,
            pltpu.SemaphoreType.DMA,
            pltpu.SemaphoreType.DMA,
            pltpu.SemaphoreType.DMA,
        ],
        compiler_params=pltpu.CompilerParams(
            use_tc_tiling_on_sc=False, needs_layout_passes=False
        ),
    )
    def body(table_hbm, idx_hbm, out_hbm, idx_v, gb0, gb1, tb0, tb1,
             g0, g1, o0, o1):
        wid = lax.axis_index("s") * NC + lax.axis_index("c")
        bt0 = wid * bt_per_w
        pltpu.sync_copy(idx_hbm.at[:, pl.ds(bt0, bt_per_w)], idx_v)

        gbuf = (gb0, gb1)
        tbuf = (tb0, tb1)
        gsem = (g0, g1)
        osem = (o0, o1)

        def fire_g(k, p):
            h = k // bt_per_w
            j = lax.rem(k, bt_per_w)
            pltpu.async_copy(table_hbm.at[idx_v.at[h, j]], gbuf[p], gsem[p])

        def drain_g(p):
            pltpu.make_async_copy(
                table_hbm.at[idx_v.at[0, 0]], gbuf[p], gsem[p]
            ).wait()

        def transpose(p):
            g, t = gbuf[p], tbuf[p]
            for grp in range(BT // 16):
                rows = lax.iota(jnp.int32, 16) + (16 * grp)
                for e in range(EMBED_DIM):
                    col = plsc.load_gather(
                        g, [rows, jnp.full((16,), e, jnp.int32)]
                    )
                    t[e // 8, e % 8, pl.ds(16 * grp, 16)] = col

        def fire_out(k, p):
            h = k // bt_per_w
            j = lax.rem(k, bt_per_w)
            pltpu.async_copy(tbuf[p], out_hbm.at[h, :, bt0 + j], osem[p])

        def wait_out(p):
            pltpu.make_async_copy(
                out_hbm.at[0, :, 0], tbuf[p], osem[p]
            ).wait()

        def half(k, p, first, last):
            if not first:
                wait_out(p)
            drain_g(p)
            transpose(p)
            if not last:
                fire_g(k + 2, p)
            fire_out(k, p)

        fire_g(0, 0)
        fire_g(1, 1)

        half(0, 0, first=True, last=False)
        half(1, 1, first=True, last=False)

        def loop_body(i, carry):
            half(2 * i, 0, first=False, last=False)
            half(2 * i + 1, 1, first=False, last=False)
            return carry

        n_pairs = n_units // 2
        lax.fori_loop(1, n_pairs - 1, loop_body, 0)

        half(n_units - 2, 0, first=False, last=True)
        half(n_units - 1, 1, first=False, last=True)
        wait_out(0)
        wait_out(1)

    return body(table, idx3)


def kernel(x, table):
    B, H = x.shape
    n_bt = B // BT
    info = plsc.get_sparse_core_info()
    NC, NS = info.num_cores, info.num_subcores
    idx3 = jnp.swapaxes(x.astype(jnp.int32), 0, 1).reshape(H, n_bt, BT)
    t5 = _sc_gather(table, idx3, H, n_bt, NC, NS)
    return jnp.transpose(t5, (2, 4, 0, 1, 3)).reshape(B, H, EMBED_DIM)


# speedup vs baseline: 1.6751x; 1.6751x over previous
import functools

import jax
import jax.numpy as jnp
from jax import lax
from jax.experimental import pallas as pl
from jax.experimental.pallas import tpu as pltpu
from jax.experimental.pallas import tpu_sc as plsc

EMBED_DIM = 32
BT = 128
ET = EMBED_DIM // 8


def _sc_gather(table, idx3, H, n_bt, NC, NS):
    NW = NC * NS
    B = n_bt * BT
    bt_per_w = n_bt // NW
    n_units = H * bt_per_w
    mesh = plsc.VectorSubcoreMesh(core_axis_name="c", subcore_axis_name="s")

    @functools.partial(
        pl.kernel,
        mesh=mesh,
        out_type=jax.ShapeDtypeStruct((H, ET, n_bt, 8, BT), jnp.float32),
        scratch_types=[
            pltpu.VMEM((H, bt_per_w, BT), jnp.int32),
            pltpu.VMEM((BT, EMBED_DIM), jnp.float32),
            pltpu.VMEM((BT, EMBED_DIM), jnp.float32),
            pltpu.VMEM((ET, 8, BT + 1), jnp.float32),
            pltpu.VMEM((ET, 8, BT + 1), jnp.float32),
            pltpu.SemaphoreType.DMA,
            pltpu.SemaphoreType.DMA,
            pltpu.SemaphoreType.DMA,
            pltpu.SemaphoreType.DMA,
        ],
        compiler_params=pltpu.CompilerParams(
            use_tc_tiling_on_sc=False, needs_layout_passes=False
        ),
    )
    def body(table_hbm, idx_hbm, out_hbm, idx_v, gb0, gb1, tb0, tb1,
             g0, g1, o0, o1):
        wid = lax.axis_index("s") * NC + lax.axis_index("c")
        bt0 = wid * bt_per_w
        pltpu.sync_copy(idx_hbm.at[:, pl.ds(bt0, bt_per_w)], idx_v)

        gbuf = (gb0, gb1)
        tbuf = (tb0, tb1)
        gsem = (g0, g1)
        osem = (o0, o1)

        def fire_g(k, p):
            h = k // bt_per_w
            j = lax.rem(k, bt_per_w)
            pltpu.async_copy(table_hbm.at[idx_v.at[h, j]], gbuf[p], gsem[p])

        def drain_g(p):
            pltpu.make_async_copy(
                table_hbm.at[idx_v.at[0, 0]], gbuf[p], gsem[p]
            ).wait()

        lanes = lax.iota(jnp.int32, 16)
        et_idx = (lanes // 8, lanes // 8 + 2)
        ei_idx = (lanes % 8, lanes % 8)
        zeros = jnp.zeros((16,), jnp.int32)

        def transpose(p):
            g, t = gbuf[p], tbuf[p]
            for b in range(BT):
                col = zeros + b
                for h in range(2):
                    v = g[b, pl.ds(16 * h, 16)]
                    plsc.store_scatter(t, [et_idx[h], ei_idx[h], col], v)

        def fire_out(k, p):
            h = k // bt_per_w
            j = lax.rem(k, bt_per_w)
            pltpu.async_copy(
                tbuf[p].at[:, :, pl.ds(0, BT)],
                out_hbm.at[h, :, bt0 + j],
                osem[p],
            )

        def wait_out(p):
            pltpu.make_async_copy(
                out_hbm.at[0, :, 0],
                tbuf[p].at[:, :, pl.ds(0, BT)],
                osem[p],
            ).wait()

        def half(k, p, first, last):
            if not first:
                wait_out(p)
            drain_g(p)
            transpose(p)
            if not last:
                fire_g(k + 2, p)
            fire_out(k, p)

        fire_g(0, 0)
        fire_g(1, 1)

        half(0, 0, first=True, last=False)
        half(1, 1, first=True, last=False)

        def loop_body(i, carry):
            half(2 * i, 0, first=False, last=False)
            half(2 * i + 1, 1, first=False, last=False)
            return carry

        n_pairs = n_units // 2
        lax.fori_loop(1, n_pairs - 1, loop_body, 0)

        half(n_units - 2, 0, first=False, last=True)
        half(n_units - 1, 1, first=False, last=True)
        wait_out(0)
        wait_out(1)

    return body(table, idx3)


def kernel(x, table):
    B, H = x.shape
    n_bt = B // BT
    info = plsc.get_sparse_core_info()
    NC, NS = info.num_cores, info.num_subcores
    idx3 = jnp.swapaxes(x.astype(jnp.int32), 0, 1).reshape(H, n_bt, BT)
    t5 = _sc_gather(table, idx3, H, n_bt, NC, NS)
    return jnp.transpose(t5, (2, 4, 0, 1, 3)).reshape(B, H, EMBED_DIM)
